# baseline (speedup 1.0000x reference)
import functools

import jax
import jax.numpy as jnp
from jax import lax
from jax.experimental import pallas as pl
from jax.experimental.pallas import tpu as pltpu
from jax.experimental.pallas import tpu_sc as plsc

P = 256
NPIX = P * P
VPAD = 1024


def _gather_bias_sc(values_padded):
    info = plsc.get_sparse_core_info()
    nc, ns, L = info.num_cores, info.num_subcores, info.num_lanes
    nw = nc * ns
    per_w = NPIX // nw
    n_vec = per_w // L

    @functools.partial(
        pl.kernel,
        mesh=plsc.VectorSubcoreMesh(core_axis_name="c", subcore_axis_name="s"),
        out_type=jax.ShapeDtypeStruct((NPIX,), jnp.float32),
        scratch_types=[
            pltpu.VMEM((VPAD,), jnp.float32),
            pltpu.VMEM((per_w,), jnp.float32),
        ],
        compiler_params=pltpu.CompilerParams(needs_layout_passes=False),
    )
    def gather_kernel(vals_hbm, out_hbm, vals_v, out_v):
        wid = lax.axis_index("s") * nc + lax.axis_index("c")
        base = wid * per_w
        pltpu.sync_copy(vals_hbm, vals_v)
        lane = lax.iota(jnp.int32, L)

        rows_w = per_w // P
        row0 = wid * rows_w
        c0 = jnp.bitwise_and(row0, 15) + 31 * lax.shift_right_logical(row0, 4) + 480

        def body(i, carry):
            ri = lax.shift_right_logical(i, 4)
            j = jnp.bitwise_and(i, 15)
            idx = (c0 + ri - 31 * j) - lane
            out_v[pl.ds(i * L, L)] = plsc.load_gather(vals_v, [idx])
            return carry

        lax.fori_loop(0, n_vec, body, 0)
        pltpu.sync_copy(out_v, out_hbm.at[pl.ds(base, per_w)])

    return gather_kernel(values_padded)


def _add_body(x_ref, b_ref, o_ref):
    o_ref[...] = x_ref[...] + b_ref[...]


def _add_tc(x3, bias2):
    rows = 32
    nsteps = x3.shape[0] // rows

    def inner(x_hbm, b_hbm, o_hbm):
        pipeline = pltpu.emit_pipeline(
            _add_body,
            grid=(nsteps,),
            in_specs=[
                pl.BlockSpec(
                    (rows, P, P),
                    lambda i: (i, 0, 0),
                    pipeline_mode=pl.Buffered(buffer_count=4),
                ),
                pl.BlockSpec((P, P), lambda i: (0, 0)),
            ],
            out_specs=[pl.BlockSpec((rows, P, P), lambda i: (i, 0, 0))],
        )
        pipeline(x_hbm, b_hbm, o_hbm)

    return pl.pallas_call(
        inner,
        in_specs=[
            pl.BlockSpec(memory_space=pl.ANY),
            pl.BlockSpec(memory_space=pl.ANY),
        ],
        out_specs=pl.BlockSpec(memory_space=pl.ANY),
        out_shape=jax.ShapeDtypeStruct(x3.shape, jnp.float32),
    )(x3, bias2)


def kernel(input, values):
    vals_padded = jnp.pad(values, (0, VPAD - values.shape[0]))
    bias = _gather_bias_sc(vals_padded).reshape(P, P)
    x3 = input.reshape(-1, P, P)
    out = _add_tc(x3, bias)
    return out.reshape(input.shape)


# speedup vs baseline: 1.0002x; 1.0002x over previous
import functools

import jax
import jax.numpy as jnp
from jax import lax
from jax.experimental import pallas as pl
from jax.experimental.pallas import tpu as pltpu
from jax.experimental.pallas import tpu_sc as plsc

P = 256
NPIX = P * P
VPAD = 1024


def _gather_bias_sc(values_padded):
    info = plsc.get_sparse_core_info()
    nc, ns, L = info.num_cores, info.num_subcores, info.num_lanes
    nw = nc * ns
    per_w = NPIX // nw
    n_vec = per_w // L

    @functools.partial(
        pl.kernel,
        mesh=plsc.VectorSubcoreMesh(core_axis_name="c", subcore_axis_name="s"),
        out_type=jax.ShapeDtypeStruct((NPIX,), jnp.float32),
        scratch_types=[
            pltpu.VMEM((VPAD,), jnp.float32),
            pltpu.VMEM((per_w,), jnp.float32),
        ],
        compiler_params=pltpu.CompilerParams(needs_layout_passes=False),
    )
    def gather_kernel(vals_hbm, out_hbm, vals_v, out_v):
        wid = lax.axis_index("s") * nc + lax.axis_index("c")
        base = wid * per_w
        pltpu.sync_copy(vals_hbm, vals_v)
        lane = lax.iota(jnp.int32, L)

        rows_w = per_w // P
        row0 = wid * rows_w
        c0 = jnp.bitwise_and(row0, 15) + 31 * lax.shift_right_logical(row0, 4) + 480

        def body(i, carry):
            ri = lax.shift_right_logical(i, 4)
            j = jnp.bitwise_and(i, 15)
            idx = (c0 + ri - 31 * j) - lane
            out_v[pl.ds(i * L, L)] = plsc.load_gather(vals_v, [idx])
            return carry

        lax.fori_loop(0, n_vec, body, 0)
        pltpu.sync_copy(out_v, out_hbm.at[pl.ds(base, per_w)])

    return gather_kernel(values_padded)


def _add_body(x_ref, b_ref, o_ref):
    o_ref[...] = x_ref[...] + b_ref[...]


def _add_tc(x3, bias2):
    rows = 8
    nsteps = x3.shape[0] // rows

    def inner(x_hbm, b_hbm, o_hbm):
        pipeline = pltpu.emit_pipeline(
            _add_body,
            grid=(nsteps,),
            in_specs=[
                pl.BlockSpec(
                    (rows, P, P),
                    lambda i: (i, 0, 0),
                    pipeline_mode=pl.Buffered(buffer_count=8),
                ),
                pl.BlockSpec((P, P), lambda i: (0, 0)),
            ],
            out_specs=[pl.BlockSpec((rows, P, P), lambda i: (i, 0, 0))],
        )
        pipeline(x_hbm, b_hbm, o_hbm)

    return pl.pallas_call(
        inner,
        in_specs=[
            pl.BlockSpec(memory_space=pl.ANY),
            pl.BlockSpec(memory_space=pl.ANY),
        ],
        out_specs=pl.BlockSpec(memory_space=pl.ANY),
        out_shape=jax.ShapeDtypeStruct(x3.shape, jnp.float32),
    )(x3, bias2)


def kernel(input, values):
    vals_padded = jnp.pad(values, (0, VPAD - values.shape[0]))
    bias = _gather_bias_sc(vals_padded).reshape(P, P)
    x3 = input.reshape(-1, P, P)
    out = _add_tc(x3, bias)
    return out.reshape(input.shape)
